# baseline (speedup 1.0000x reference)
import functools

import jax
import jax.numpy as jnp
from jax import lax
from jax.experimental import pallas as pl
from jax.experimental.pallas import tpu as pltpu
from jax.experimental.pallas import tpu_sc as plsc

BATCH = 16384
EMBED = 64
HIDDEN = 128

NUM_CORES = 2
NUM_SUBCORES = 16
NUM_WORKERS = NUM_CORES * NUM_SUBCORES
B_PER_W = BATCH // NUM_WORKERS
IDX_CHUNK = 128
NCHUNK = B_PER_W // IDX_CHUNK


def _sc_gather_add(user_table, movie_table, users2d, movies2d):
    mesh = plsc.VectorSubcoreMesh(core_axis_name="c", subcore_axis_name="s",
                                  num_cores=NUM_CORES,
                                  num_subcores=NUM_SUBCORES)

    @functools.partial(
        pl.kernel,
        out_type=jax.ShapeDtypeStruct((BATCH, EMBED), jnp.float32),
        mesh=mesh,
        scratch_types=[
            pltpu.VMEM((NCHUNK, IDX_CHUNK), jnp.int32),
            pltpu.VMEM((NCHUNK, IDX_CHUNK), jnp.int32),
            pltpu.VMEM((B_PER_W, EMBED), jnp.float32),
            pltpu.VMEM((B_PER_W, EMBED), jnp.float32),
            pltpu.SemaphoreType.DMA,
        ],
        compiler_params=pltpu.CompilerParams(use_tc_tiling_on_sc=False),
    )
    def kern(ut_hbm, mt_hbm, u_hbm, m_hbm, out_hbm, idx_u, idx_m, rows_u,
             rows_m, sem):
        wid = lax.axis_index("s") * NUM_CORES + lax.axis_index("c")
        base = wid * B_PER_W
        pltpu.sync_copy(u_hbm.at[pl.ds(wid * NCHUNK, NCHUNK)], idx_u)
        pltpu.sync_copy(m_hbm.at[pl.ds(wid * NCHUNK, NCHUNK)], idx_m)
        copies = []
        for j in range(NCHUNK):
            copies.append(pltpu.async_copy(
                ut_hbm.at[idx_u.at[j]],
                rows_u.at[pl.ds(j * IDX_CHUNK, IDX_CHUNK)], sem))
            copies.append(pltpu.async_copy(
                mt_hbm.at[idx_m.at[j]],
                rows_m.at[pl.ds(j * IDX_CHUNK, IDX_CHUNK)], sem))
        for c in copies:
            c.wait()

        def body(r, _):
            for c in range(EMBED // 16):
                sl = pl.ds(c * 16, 16)
                rows_u[r, sl] = rows_u[r, sl] + rows_m[r, sl]
            return 0

        lax.fori_loop(0, B_PER_W, body, 0, unroll=2)
        pltpu.sync_copy(rows_u, out_hbm.at[pl.ds(base, B_PER_W)])

    return kern(user_table, movie_table, users2d, movies2d)


def _mlp_block(x_ref, w1_ref, b1_ref, w2_ref, b2_ref, o_ref):
    x = x_ref[...]
    h = jnp.maximum(
        jnp.dot(x, w1_ref[...], preferred_element_type=jnp.float32)
        + b1_ref[...], 0.0)
    o_ref[...] = (jnp.sum(h * w2_ref[...], axis=1, keepdims=True)
                  + b2_ref[0, 0])


def _tc_mlp(combined, W1, b1, W2, b2):
    nblk = 16
    blk = BATCH // nblk
    return pl.pallas_call(
        _mlp_block,
        grid=(nblk,),
        in_specs=[
            pl.BlockSpec((blk, EMBED), lambda i: (i, 0)),
            pl.BlockSpec((EMBED, HIDDEN), lambda i: (0, 0)),
            pl.BlockSpec((1, HIDDEN), lambda i: (0, 0)),
            pl.BlockSpec((1, HIDDEN), lambda i: (0, 0)),
            pl.BlockSpec((1, 1), lambda i: (0, 0)),
        ],
        out_specs=pl.BlockSpec((blk, 1), lambda i: (i, 0)),
        out_shape=jax.ShapeDtypeStruct((BATCH, 1), jnp.float32),
    )(combined, W1, b1.reshape(1, HIDDEN), W2.reshape(1, HIDDEN),
      b2.reshape(1, 1))


@jax.jit
def kernel(users, movies, user_table, movie_table, W1, b1, W2, b2):
    users2d = users.astype(jnp.int32).reshape(NUM_WORKERS * NCHUNK, IDX_CHUNK)
    movies2d = movies.astype(jnp.int32).reshape(NUM_WORKERS * NCHUNK,
                                                IDX_CHUNK)
    combined = _sc_gather_add(user_table, movie_table, users2d, movies2d)
    return _tc_mlp(combined, W1, b1, W2, b2)


# speedup vs baseline: 1.8917x; 1.8917x over previous
import functools

import jax
import jax.numpy as jnp
from jax import lax
from jax.experimental import pallas as pl
from jax.experimental.pallas import tpu as pltpu
from jax.experimental.pallas import tpu_sc as plsc

BATCH = 16384
EMBED = 64
HIDDEN = 128

NUM_CORES = 2
NUM_SUBCORES = 16
NUM_WORKERS = NUM_CORES * NUM_SUBCORES
B_PER_W = BATCH // NUM_WORKERS
GRP = 2
CHUNK = 64
NGRP = CHUNK // GRP
NCHUNKS = B_PER_W // CHUNK


def _sc_gather_add(ut_t, mt_t, users, movies):
    mesh = plsc.VectorSubcoreMesh(core_axis_name="c", subcore_axis_name="s",
                                  num_cores=NUM_CORES,
                                  num_subcores=NUM_SUBCORES)

    @functools.partial(
        pl.kernel,
        out_type=jax.ShapeDtypeStruct((BATCH, EMBED), jnp.float32),
        mesh=mesh,
        scratch_types=[
            pltpu.VMEM((B_PER_W + 16,), jnp.int32),
            pltpu.VMEM((B_PER_W + 16,), jnp.int32),
            pltpu.VMEM((GRP, EMBED, 128), jnp.float32),
            pltpu.VMEM((GRP, EMBED, 128), jnp.float32),
            pltpu.VMEM((CHUNK, EMBED), jnp.float32),
            pltpu.SemaphoreType.DMA,
        ],
        compiler_params=pltpu.CompilerParams(needs_layout_passes=False),
    )
    def kern(ut_hbm, mt_hbm, u_hbm, m_hbm, out_hbm, idx_u, idx_m, uslab,
             mslab, comb, sem):
        wid = lax.axis_index("s") * NUM_CORES + lax.axis_index("c")
        base = wid * B_PER_W
        pltpu.sync_copy(u_hbm.at[pl.ds(base, B_PER_W)],
                        idx_u.at[pl.ds(0, B_PER_W)])
        pltpu.sync_copy(m_hbm.at[pl.ds(base, B_PER_W)],
                        idx_m.at[pl.ds(0, B_PER_W)])
        rows = lax.iota(jnp.int32, 16)

        def chunk_body(ch, _):
            def grp_body(k, _):
                off = ch * CHUNK + k * GRP
                vu = idx_u[pl.ds(off, 16)]
                vm = idx_m[pl.ds(off, 16)]
                copies = []
                for j in range(GRP):
                    ou = pl.multiple_of((vu[j] >> 7) * 128, 128)
                    om = pl.multiple_of((vm[j] >> 7) * 128, 128)
                    copies.append(pltpu.async_copy(
                        ut_hbm.at[:, pl.ds(ou, 128)], uslab.at[j], sem))
                    copies.append(pltpu.async_copy(
                        mt_hbm.at[:, pl.ds(om, 128)], mslab.at[j], sem))
                for c in copies:
                    c.wait()
                for j in range(GRP):
                    cu = jnp.full((16,), vu[j] & 127, jnp.int32)
                    cm = jnp.full((16,), vm[j] & 127, jnp.int32)
                    js = jnp.full((16,), j, jnp.int32)
                    for g in range(EMBED // 16):
                        rg = rows + g * 16
                        eu = plsc.load_gather(uslab, [js, rg, cu])
                        em = plsc.load_gather(mslab, [js, rg, cm])
                        comb[k * GRP + j, pl.ds(g * 16, 16)] = eu + em
                return 0

            lax.fori_loop(0, NGRP, grp_body, 0)
            dst = pl.multiple_of(base + ch * CHUNK, CHUNK)
            pltpu.sync_copy(comb, out_hbm.at[pl.ds(dst, CHUNK)])
            return 0

        lax.fori_loop(0, NCHUNKS, chunk_body, 0)

    return kern(ut_t, mt_t, users, movies)


def _mlp_block(x_ref, w1_ref, b1_ref, w2_ref, b2_ref, o_ref):
    x = x_ref[...]
    h = jnp.maximum(
        jnp.dot(x, w1_ref[...], preferred_element_type=jnp.float32)
        + b1_ref[...], 0.0)
    o_ref[...] = (jnp.sum(h * w2_ref[...], axis=1, keepdims=True)
                  + b2_ref[0, 0])


def _tc_mlp(combined, W1, b1, W2, b2):
    nblk = 16
    blk = BATCH // nblk
    return pl.pallas_call(
        _mlp_block,
        grid=(nblk,),
        in_specs=[
            pl.BlockSpec((blk, EMBED), lambda i: (i, 0)),
            pl.BlockSpec((EMBED, HIDDEN), lambda i: (0, 0)),
            pl.BlockSpec((1, HIDDEN), lambda i: (0, 0)),
            pl.BlockSpec((1, HIDDEN), lambda i: (0, 0)),
            pl.BlockSpec((1, 1), lambda i: (0, 0)),
        ],
        out_specs=pl.BlockSpec((blk, 1), lambda i: (i, 0)),
        out_shape=jax.ShapeDtypeStruct((BATCH, 1), jnp.float32),
    )(combined, W1, b1.reshape(1, HIDDEN), W2.reshape(1, HIDDEN),
      b2.reshape(1, 1))


@jax.jit
def kernel(users, movies, user_table, movie_table, W1, b1, W2, b2):
    ut_t = jnp.transpose(user_table)
    mt_t = jnp.transpose(movie_table)
    combined = _sc_gather_add(ut_t, mt_t, users.astype(jnp.int32),
                              movies.astype(jnp.int32))
    return _tc_mlp(combined, W1, b1, W2, b2)


# speedup vs baseline: 2.1121x; 1.1165x over previous
import functools

import jax
import jax.numpy as jnp
from jax import lax
from jax.experimental import pallas as pl
from jax.experimental.pallas import tpu as pltpu
from jax.experimental.pallas import tpu_sc as plsc

BATCH = 16384
EMBED = 64
HIDDEN = 128

NUM_CORES = 2
NUM_SUBCORES = 16
NUM_WORKERS = NUM_CORES * NUM_SUBCORES
B_PER_W = BATCH // NUM_WORKERS
CHUNK = 64
NCHUNKS = B_PER_W // CHUNK
NPAIR = CHUNK // 2


def _sc_gather_add(ut_t, mt_t, users, movies):
    mesh = plsc.VectorSubcoreMesh(core_axis_name="c", subcore_axis_name="s",
                                  num_cores=NUM_CORES,
                                  num_subcores=NUM_SUBCORES)

    @functools.partial(
        pl.kernel,
        out_type=jax.ShapeDtypeStruct((BATCH, EMBED), jnp.float32),
        mesh=mesh,
        scratch_types=[
            pltpu.VMEM((B_PER_W + 16,), jnp.int32),
            pltpu.VMEM((B_PER_W + 16,), jnp.int32),
            pltpu.VMEM((2, EMBED, 128), jnp.float32),
            pltpu.VMEM((2, EMBED, 128), jnp.float32),
            pltpu.VMEM((CHUNK, EMBED), jnp.float32),
            pltpu.SemaphoreType.DMA,
            pltpu.SemaphoreType.DMA,
        ],
        compiler_params=pltpu.CompilerParams(needs_layout_passes=False),
    )
    def kern(ut_hbm, mt_hbm, u_hbm, m_hbm, out_hbm, idx_u, idx_m, uslab,
             mslab, comb, sem0, sem1):
        wid = lax.axis_index("s") * NUM_CORES + lax.axis_index("c")
        base = wid * B_PER_W
        pltpu.sync_copy(u_hbm.at[pl.ds(base, B_PER_W)],
                        idx_u.at[pl.ds(0, B_PER_W)])
        pltpu.sync_copy(m_hbm.at[pl.ds(base, B_PER_W)],
                        idx_m.at[pl.ds(0, B_PER_W)])
        rows = lax.iota(jnp.int32, 16)
        sems = (sem0, sem1)

        def fire(i, p, guard):
            def _go():
                ru = idx_u[pl.ds(i, 16)][0]
                rm = idx_m[pl.ds(i, 16)][0]
                ou = pl.multiple_of((ru >> 7) * 128, 128)
                om = pl.multiple_of((rm >> 7) * 128, 128)
                pltpu.async_copy(ut_hbm.at[:, pl.ds(ou, 128)],
                                 uslab.at[p], sems[p])
                pltpu.async_copy(mt_hbm.at[:, pl.ds(om, 128)],
                                 mslab.at[p], sems[p])
            if guard is None:
                _go()
            else:
                pl.when(guard)(_go)

        def wait_and_extract(i, row, p):
            pltpu.make_async_copy(ut_hbm.at[:, pl.ds(0, 128)],
                                  uslab.at[p], sems[p]).wait()
            pltpu.make_async_copy(mt_hbm.at[:, pl.ds(0, 128)],
                                  mslab.at[p], sems[p]).wait()
            ru = idx_u[pl.ds(i, 16)][0]
            rm = idx_m[pl.ds(i, 16)][0]
            cu = jnp.full((16,), ru & 127, jnp.int32)
            cm = jnp.full((16,), rm & 127, jnp.int32)
            ps = jnp.full((16,), p, jnp.int32)
            for g in range(EMBED // 16):
                rg = rows + g * 16
                eu = plsc.load_gather(uslab, [ps, rg, cu])
                em = plsc.load_gather(mslab, [ps, rg, cm])
                comb[row, pl.ds(g * 16, 16)] = eu + em

        fire(0, 0, None)
        fire(1, 1, None)

        def chunk_body(ch, _):
            def pair_body(q, _):
                i0 = ch * CHUNK + q * 2
                wait_and_extract(i0, q * 2, 0)
                fire(i0 + 2, 0, i0 + 2 < B_PER_W)
                wait_and_extract(i0 + 1, q * 2 + 1, 1)
                fire(i0 + 3, 1, i0 + 3 < B_PER_W)
                return 0

            lax.fori_loop(0, NPAIR, pair_body, 0)
            dst = pl.multiple_of(base + ch * CHUNK, CHUNK)
            pltpu.sync_copy(comb, out_hbm.at[pl.ds(dst, CHUNK)])
            return 0

        lax.fori_loop(0, NCHUNKS, chunk_body, 0)

    return kern(ut_t, mt_t, users, movies)


def _mlp_block(x_ref, w1_ref, b1_ref, w2_ref, b2_ref, o_ref):
    x = x_ref[...]
    h = jnp.maximum(
        jnp.dot(x, w1_ref[...], preferred_element_type=jnp.float32)
        + b1_ref[...], 0.0)
    o_ref[...] = (jnp.sum(h * w2_ref[...], axis=1, keepdims=True)
                  + b2_ref[0, 0])


def _tc_mlp(combined, W1, b1, W2, b2):
    nblk = 16
    blk = BATCH // nblk
    return pl.pallas_call(
        _mlp_block,
        grid=(nblk,),
        in_specs=[
            pl.BlockSpec((blk, EMBED), lambda i: (i, 0)),
            pl.BlockSpec((EMBED, HIDDEN), lambda i: (0, 0)),
            pl.BlockSpec((1, HIDDEN), lambda i: (0, 0)),
            pl.BlockSpec((1, HIDDEN), lambda i: (0, 0)),
            pl.BlockSpec((1, 1), lambda i: (0, 0)),
        ],
        out_specs=pl.BlockSpec((blk, 1), lambda i: (i, 0)),
        out_shape=jax.ShapeDtypeStruct((BATCH, 1), jnp.float32),
    )(combined, W1, b1.reshape(1, HIDDEN), W2.reshape(1, HIDDEN),
      b2.reshape(1, 1))


@jax.jit
def kernel(users, movies, user_table, movie_table, W1, b1, W2, b2):
    ut_t = jnp.transpose(user_table)
    mt_t = jnp.transpose(movie_table)
    combined = _sc_gather_add(ut_t, mt_t, users.astype(jnp.int32),
                              movies.astype(jnp.int32))
    return _tc_mlp(combined, W1, b1, W2, b2)


# speedup vs baseline: 2.3842x; 1.1288x over previous
import functools

import jax
import jax.numpy as jnp
from jax import lax
from jax.experimental import pallas as pl
from jax.experimental.pallas import tpu as pltpu
from jax.experimental.pallas import tpu_sc as plsc

BATCH = 16384
EMBED = 64
HIDDEN = 128

NUM_CORES = 2
NUM_SUBCORES = 16
B_PER_W = BATCH // NUM_SUBCORES
CHUNK = 64
NCHUNKS = B_PER_W // CHUNK
NSETS = 4
NQ = CHUNK // NSETS


def _sc_gather(ut_t, mt_t, idx_all):
    mesh = plsc.VectorSubcoreMesh(core_axis_name="c", subcore_axis_name="s",
                                  num_cores=NUM_CORES,
                                  num_subcores=NUM_SUBCORES)

    @functools.partial(
        pl.kernel,
        out_type=jax.ShapeDtypeStruct((NUM_CORES, BATCH, EMBED),
                                      jnp.float32),
        mesh=mesh,
        scratch_types=[
            pltpu.VMEM((B_PER_W + 16,), jnp.int32),
            pltpu.VMEM((NSETS, EMBED, 128), jnp.float32),
            pltpu.VMEM((CHUNK, EMBED), jnp.float32),
            pltpu.SemaphoreType.DMA,
            pltpu.SemaphoreType.DMA,
            pltpu.SemaphoreType.DMA,
            pltpu.SemaphoreType.DMA,
        ],
        compiler_params=pltpu.CompilerParams(needs_layout_passes=False),
    )
    def kern(ut_hbm, mt_hbm, idx_hbm, out_hbm, idx, slab, comb, s0, s1, s2,
             s3):
        c = lax.axis_index("c")
        s = lax.axis_index("s")
        base = s * B_PER_W
        sems = (s0, s1, s2, s3)
        pltpu.sync_copy(idx_hbm.at[c, pl.ds(base, B_PER_W)],
                        idx.at[pl.ds(0, B_PER_W)])
        rows = lax.iota(jnp.int32, 16)

        def fire(i, p, guard):
            def _u():
                r = idx[pl.ds(i, 16)][0]
                off = pl.multiple_of((r >> 7) * 128, 128)
                pltpu.async_copy(ut_hbm.at[:, pl.ds(off, 128)],
                                 slab.at[p], sems[p])

            def _m():
                r = idx[pl.ds(i, 16)][0]
                off = pl.multiple_of((r >> 7) * 128, 128)
                pltpu.async_copy(mt_hbm.at[:, pl.ds(off, 128)],
                                 slab.at[p], sems[p])

            g = True if guard is None else guard
            pl.when(jnp.logical_and(c == 0, g))(_u)
            pl.when(jnp.logical_and(c == 1, g))(_m)

        def wait_and_extract(i, row, p):
            pltpu.make_async_copy(ut_hbm.at[:, pl.ds(0, 128)],
                                  slab.at[p], sems[p]).wait()
            r = idx[pl.ds(i, 16)][0]
            cv = jnp.full((16,), r & 127, jnp.int32)
            ps = jnp.full((16,), p, jnp.int32)
            for g in range(EMBED // 16):
                e = plsc.load_gather(slab, [ps, rows + g * 16, cv])
                comb[row, pl.ds(g * 16, 16)] = e

        for p in range(NSETS):
            fire(p, p, None)

        def chunk_body(ch, _):
            def q_body(q, _):
                i0 = ch * CHUNK + q * NSETS
                for j in range(NSETS):
                    wait_and_extract(i0 + j, q * NSETS + j, j)
                    fire(i0 + j + NSETS, j, i0 + j + NSETS < B_PER_W)
                return 0

            lax.fori_loop(0, NQ, q_body, 0)
            dst = pl.multiple_of(base + ch * CHUNK, CHUNK)
            pltpu.sync_copy(comb, out_hbm.at[c, pl.ds(dst, CHUNK)])
            return 0

        lax.fori_loop(0, NCHUNKS, chunk_body, 0)

    return kern(ut_t, mt_t, idx_all)


def _mlp_block(u_ref, m_ref, w1_ref, b1_ref, w2_ref, b2_ref, o_ref):
    x = u_ref[0] + m_ref[0]
    h = jnp.maximum(
        jnp.dot(x, w1_ref[...], preferred_element_type=jnp.float32)
        + b1_ref[...], 0.0)
    o_ref[...] = (jnp.sum(h * w2_ref[...], axis=1, keepdims=True)
                  + b2_ref[0, 0])


def _tc_mlp(emb, W1, b1, W2, b2):
    nblk = 16
    blk = BATCH // nblk
    return pl.pallas_call(
        _mlp_block,
        grid=(nblk,),
        in_specs=[
            pl.BlockSpec((1, blk, EMBED), lambda i: (0, i, 0)),
            pl.BlockSpec((1, blk, EMBED), lambda i: (1, i, 0)),
            pl.BlockSpec((EMBED, HIDDEN), lambda i: (0, 0)),
            pl.BlockSpec((1, HIDDEN), lambda i: (0, 0)),
            pl.BlockSpec((1, HIDDEN), lambda i: (0, 0)),
            pl.BlockSpec((1, 1), lambda i: (0, 0)),
        ],
        out_specs=pl.BlockSpec((blk, 1), lambda i: (i, 0)),
        out_shape=jax.ShapeDtypeStruct((BATCH, 1), jnp.float32),
    )(emb, emb, W1, b1.reshape(1, HIDDEN), W2.reshape(1, HIDDEN),
      b2.reshape(1, 1))


@jax.jit
def kernel(users, movies, user_table, movie_table, W1, b1, W2, b2):
    ut_t = jnp.transpose(user_table)
    mt_t = jnp.transpose(movie_table)
    idx_all = jnp.stack([users.astype(jnp.int32), movies.astype(jnp.int32)])
    emb = _sc_gather(ut_t, mt_t, idx_all)
    return _tc_mlp(emb, W1, b1, W2, b2)
